# baseline (speedup 1.0000x reference)
import jax
import jax.numpy as jnp
import numpy as np
from jax.experimental import pallas as pl
from jax.experimental.pallas import tpu as pltpu

B = 32
N = 1000000
BLK = 2048
CH = 256
NCH = BLK // CH
NB = (N + BLK - 1) // BLK

_TINY = np.float32(np.finfo(np.float32).tiny)
_K0 = np.uint32(0)
_K1 = np.uint32(42)
_K2 = np.uint32(np.uint32(0) ^ np.uint32(42) ^ np.uint32(0x1BD11BDA))
_KS = (_K0, _K1, _K2)
_ROT = ((13, 15, 26, 6), (17, 29, 16, 24))
_INJ = ((1, 2), (2, 0), (0, 1), (1, 2), (2, 0))


def _rotl(x, r):
    return (x << np.uint32(r)) | (x >> np.uint32(32 - r))


def _threefry_bits(cnt):
    x1i = cnt + _KS[1]
    x0 = x1i
    x1 = _rotl(x1i, _ROT[0][0]) ^ x1i
    first = True
    for g in range(5):
        for r in _ROT[g % 2]:
            if first:
                first = False
                continue
            x0 = x0 + x1
            x1 = _rotl(x1, r) ^ x0
        a, b = _INJ[g]
        x0 = x0 + _KS[a]
        x1 = x1 + (_KS[b] + np.uint32(g + 1))
    return x0 ^ x1


def _gumbel_from_bits(bits):
    fb = (bits >> np.uint32(9)) | np.uint32(0x3F800000)
    fl = jax.lax.bitcast_convert_type(fb, jnp.float32) - jnp.float32(1.0)
    u = jnp.maximum(fl, _TINY)
    return -jnp.log(-jnp.log(u))


def _body(x_ref, act_ref, nlp_ref,
          pmax_ref, idx_ref, pick_ref, m_ref, s_ref):
    c = pl.program_id(0)

    @pl.when(c == 0)
    def _init():
        pmax_ref[...] = jnp.full((B,), -jnp.inf, jnp.float32)
        idx_ref[...] = jnp.zeros((B,), jnp.int32)
        pick_ref[...] = jnp.zeros((B,), jnp.float32)
        m_ref[...] = jnp.full((B,), -jnp.inf, jnp.float32)
        s_ref[...] = jnp.zeros((B,), jnp.float32)

    base = c * BLK
    pm = pmax_ref[...]
    ix = idx_ref[...]
    pk = pick_ref[...]
    m = m_ref[...]
    s = s_ref[...]

    row_flat = jax.lax.broadcasted_iota(jnp.uint32, (B, CH), 0) * np.uint32(N)
    ch_iota = jax.lax.broadcasted_iota(jnp.int32, (B, CH), 1)

    for k in range(NCH):
        x = x_ref[:, k * CH:(k + 1) * CH]
        col = (base + k * CH) + ch_iota
        valid = col < N
        flat = row_flat + col.astype(jnp.uint32)
        bits = _threefry_bits(flat)
        g = _gumbel_from_bits(bits)

        p = jnp.where(valid, x + g, -jnp.inf)
        bpm = jnp.max(p, axis=1)
        cand = jnp.where(p == bpm[:, None], col, jnp.int32(2**30))
        bidx = jnp.min(cand, axis=1)
        bpick = jnp.max(jnp.where(col == bidx[:, None], x, -jnp.inf), axis=1)

        xm = jnp.where(valid, x, -jnp.inf)
        bm = jnp.max(xm, axis=1)
        bmc = jnp.maximum(bm, jnp.float32(-1e30))
        bs = jnp.sum(jnp.exp(xm - bmc[:, None]), axis=1)

        better = bpm > pm
        pm = jnp.where(better, bpm, pm)
        ix = jnp.where(better, bidx, ix)
        pk = jnp.where(better, bpick, pk)

        mn = jnp.maximum(m, bm)
        s = s * jnp.exp(m - mn) + bs * jnp.exp(bm - mn)
        m = mn

    pmax_ref[...] = pm
    idx_ref[...] = ix
    pick_ref[...] = pk
    m_ref[...] = m
    s_ref[...] = s

    @pl.when(c == NB - 1)
    def _fin():
        act_ref[...] = idx_ref[...]
        nlp_ref[...] = (m_ref[...] + jnp.log(s_ref[...])) - pick_ref[...]


@jax.jit
def kernel(logits):
    action, neglogprob = pl.pallas_call(
        _body,
        grid=(NB,),
        in_specs=[pl.BlockSpec((B, BLK), lambda c: (0, c))],
        out_specs=[
            pl.BlockSpec((B,), lambda c: (0,)),
            pl.BlockSpec((B,), lambda c: (0,)),
        ],
        out_shape=[
            jax.ShapeDtypeStruct((B,), jnp.int32),
            jax.ShapeDtypeStruct((B,), jnp.float32),
        ],
        scratch_shapes=[
            pltpu.VMEM((B,), jnp.float32),
            pltpu.VMEM((B,), jnp.int32),
            pltpu.VMEM((B,), jnp.float32),
            pltpu.VMEM((B,), jnp.float32),
            pltpu.VMEM((B,), jnp.float32),
        ],
    )(logits)
    return action, neglogprob


# speedup vs baseline: 1.8960x; 1.8960x over previous
import jax
import jax.numpy as jnp
import numpy as np
from jax.experimental import pallas as pl
from jax.experimental.pallas import tpu as pltpu

B = 32
N = 1000000
BLK = 2048
CH = 256
NCH = BLK // CH
NB = (N + BLK - 1) // BLK
NFULL = NB - 1
TAIL = N - NFULL * BLK
NCH_TAIL = (TAIL + CH - 1) // CH

_TINY = np.float32(np.finfo(np.float32).tiny)
_NEG = np.float32(-3.0e38)
_K0 = np.uint32(0)
_K1 = np.uint32(42)
_K2 = np.uint32(np.uint32(0) ^ np.uint32(42) ^ np.uint32(0x1BD11BDA))
_KS = (_K0, _K1, _K2)
_ROT = ((13, 15, 26, 6), (17, 29, 16, 24))
_INJ = ((1, 2), (2, 0), (0, 1), (1, 2), (2, 0))


def _rotl(x, r):
    return (x << np.uint32(r)) | (x >> np.uint32(32 - r))


def _threefry_bits(cnt):
    x1i = cnt + _KS[1]
    x0 = x1i
    x1 = _rotl(x1i, _ROT[0][0]) ^ x1i
    first = True
    for g in range(5):
        for r in _ROT[g % 2]:
            if first:
                first = False
                continue
            x0 = x0 + x1
            x1 = _rotl(x1, r) ^ x0
        a, b = _INJ[g]
        x0 = x0 + _KS[a]
        x1 = x1 + (_KS[b] + np.uint32(g + 1))
    return x0 ^ x1


def _gumbel_from_bits(bits):
    fb = (bits >> np.uint32(9)) | np.uint32(0x3F800000)
    fl = jax.lax.bitcast_convert_type(fb, jnp.float32) - jnp.float32(1.0)
    u = jnp.maximum(fl, _TINY)
    return -jnp.log(-jnp.log(u))


def _chunk_update(x, col, masked,
                  pmax_ref, pidx_ref, pick_ref, m_ref, s_ref, sl):
    flat = (jax.lax.broadcasted_iota(jnp.uint32, (B, CH), 0) * np.uint32(N)
            + col.astype(jnp.uint32))
    g = _gumbel_from_bits(_threefry_bits(flat))
    p = x + g
    if masked:
        valid = col < N
        p = jnp.where(valid, p, -jnp.inf)
        xs = jnp.where(valid, x, -jnp.inf)
    else:
        xs = x

    pm = pmax_ref[:, sl]
    upd = p > pm
    pmax_ref[:, sl] = jnp.where(upd, p, pm)
    pidx_ref[:, sl] = jnp.where(upd, col, pidx_ref[:, sl])
    pick_ref[:, sl] = jnp.where(upd, x, pick_ref[:, sl])

    mo = m_ref[:, sl]
    nm = jnp.maximum(mo, xs)
    s_ref[:, sl] = s_ref[:, sl] * jnp.exp(mo - nm) + jnp.exp(xs - nm)
    m_ref[:, sl] = nm


def _body(x_ref, act_ref, nlp_ref,
          pmax_ref, pidx_ref, pick_ref, m_ref, s_ref):
    c = pl.program_id(0)

    @pl.when(c == 0)
    def _init():
        pmax_ref[...] = jnp.full((B, BLK), -jnp.inf, jnp.float32)
        pidx_ref[...] = jnp.zeros((B, BLK), jnp.int32)
        pick_ref[...] = jnp.zeros((B, BLK), jnp.float32)
        m_ref[...] = jnp.full((B, BLK), -jnp.inf, jnp.float32)
        s_ref[...] = jnp.zeros((B, BLK), jnp.float32)

    base = c * BLK
    ch_iota = jax.lax.broadcasted_iota(jnp.int32, (B, CH), 1)

    @pl.when(c < NFULL)
    def _full():
        for k in range(NCH):
            sl = slice(k * CH, (k + 1) * CH)
            col = (base + k * CH) + ch_iota
            _chunk_update(x_ref[:, sl], col, False,
                          pmax_ref, pidx_ref, pick_ref, m_ref, s_ref, sl)

    @pl.when(c == NFULL)
    def _tail():
        for k in range(NCH_TAIL):
            sl = slice(k * CH, (k + 1) * CH)
            col = (base + k * CH) + ch_iota
            _chunk_update(x_ref[:, sl], col, True,
                          pmax_ref, pidx_ref, pick_ref, m_ref, s_ref, sl)

        pmax_v = pmax_ref[...]
        pidx_v = pidx_ref[...]
        pick_v = pick_ref[...]
        m_v = m_ref[...]
        s_v = s_ref[...]

        bpm = jnp.max(pmax_v, axis=1)
        winners = pmax_v == bpm[:, None]
        idx = jnp.min(jnp.where(winners, pidx_v, jnp.int32(2**30)), axis=1)
        atidx = pidx_v == idx[:, None]
        pick = jnp.max(jnp.where(atidx, pick_v, -jnp.inf), axis=1)

        mf = jnp.max(m_v, axis=1)
        z = jnp.sum(s_v * jnp.exp(m_v - mf[:, None]), axis=1)

        act_ref[...] = idx
        nlp_ref[...] = (mf + jnp.log(z)) - pick


@jax.jit
def kernel(logits):
    action, neglogprob = pl.pallas_call(
        _body,
        grid=(NB,),
        in_specs=[pl.BlockSpec((B, BLK), lambda c: (0, c))],
        out_specs=[
            pl.BlockSpec((B,), lambda c: (0,)),
            pl.BlockSpec((B,), lambda c: (0,)),
        ],
        out_shape=[
            jax.ShapeDtypeStruct((B,), jnp.int32),
            jax.ShapeDtypeStruct((B,), jnp.float32),
        ],
        scratch_shapes=[
            pltpu.VMEM((B, BLK), jnp.float32),
            pltpu.VMEM((B, BLK), jnp.int32),
            pltpu.VMEM((B, BLK), jnp.float32),
            pltpu.VMEM((B, BLK), jnp.float32),
            pltpu.VMEM((B, BLK), jnp.float32),
        ],
    )(logits)
    return action, neglogprob
